# baseline (speedup 1.0000x reference)
import functools

import jax
import jax.numpy as jnp
from jax import lax
from jax.experimental import pallas as pl
from jax.experimental.pallas import tpu as pltpu
from jax.experimental.pallas import tpu_sc as plsc

_B = 4096
_L = 200
_D = 64
_NW = 32
_CHUNK = 128
_N = _B * _L
_PER_W = _N // _NW
_NCH = _PER_W // _CHUNK
_VPR = _D // 16


def _make_sc_call():
  mesh = plsc.VectorSubcoreMesh(core_axis_name="c", subcore_axis_name="s")

  @functools.partial(
      pl.kernel,
      out_type=jax.ShapeDtypeStruct((_N, _D), jnp.float32),
      mesh=mesh,
      compiler_params=pltpu.CompilerParams(use_tc_tiling_on_sc=False),
      scratch_types=[
          pltpu.VMEM((_NCH, _CHUNK), jnp.int32),
          pltpu.VMEM((_L * _D,), jnp.float32),
          pltpu.VMEM((_CHUNK, _D), jnp.float32),
          pltpu.VMEM((_CHUNK, _D), jnp.float32),
          pltpu.SemaphoreType.DMA,
          pltpu.SemaphoreType.DMA,
      ],
  )
  def sc_embed(x_hbm, tok_hbm, pos_hbm, out_hbm,
               idx_v, pos_v, rows0, rows1, sem0, sem1):
    nc = 2
    wid = lax.axis_index("s") * nc + lax.axis_index("c")
    row_base = wid * _PER_W

    pltpu.sync_copy(x_hbm.at[pl.ds(wid * _NCH, _NCH)], idx_v)
    pltpu.sync_copy(pos_hbm, pos_v)

    def add_pos(rows, po):
      def row_body(i, po):
        off = po * _D
        for j in range(_VPR):
          pv = pos_v[pl.ds(off + j * 16, 16)]
          plsc.addupdate(rows.at[i, pl.ds(j * 16, 16)], pv)
        po = po + 1
        return lax.select(po == _L, 0, po)
      return lax.fori_loop(0, _CHUNK, row_body, po, unroll=2)

    pltpu.async_copy(tok_hbm.at[idx_v.at[0]], rows0, sem0)

    def pair_body(g, po):
      c0 = 2 * g
      pltpu.async_copy(tok_hbm.at[idx_v.at[c0 + 1]], rows1, sem1)
      pltpu.make_async_copy(tok_hbm.at[idx_v.at[c0]], rows0, sem0).wait()
      po = add_pos(rows0, po)
      pltpu.sync_copy(rows0, out_hbm.at[pl.ds(row_base + c0 * _CHUNK, _CHUNK)])

      @pl.when(c0 + 2 < _NCH)
      def _():
        pltpu.async_copy(tok_hbm.at[idx_v.at[c0 + 2]], rows0, sem0)

      pltpu.make_async_copy(tok_hbm.at[idx_v.at[c0 + 1]], rows1, sem1).wait()
      po = add_pos(rows1, po)
      pltpu.sync_copy(
          rows1, out_hbm.at[pl.ds(row_base + (c0 + 1) * _CHUNK, _CHUNK)])
      return po

    lax.fori_loop(0, _NCH // 2, pair_body, 0)

  return sc_embed


_sc_embed = _make_sc_call()


@jax.jit
def kernel(x, tok_emb, pos_emb):
  x2d = x.astype(jnp.int32).reshape(_N // _CHUNK, _CHUNK)
  pos_flat = pos_emb[:_L].reshape(-1)
  out = _sc_embed(x2d, tok_emb, pos_flat)
  return out.reshape(_B, _L, _D)


# speedup vs baseline: 1.0106x; 1.0106x over previous
import functools

import jax
import jax.numpy as jnp
from jax import lax
from jax.experimental import pallas as pl
from jax.experimental.pallas import tpu as pltpu
from jax.experimental.pallas import tpu_sc as plsc

_B = 4096
_L = 200
_D = 64
_NW = 32
_CHUNK = 128
_N = _B * _L
_PER_W = _N // _NW
_NCH = _PER_W // _CHUNK
_NSLOT = 8
_LEAD = 4
_VPC = _CHUNK * _D // 16


def _make_sc_call():
  mesh = plsc.VectorSubcoreMesh(core_axis_name="c", subcore_axis_name="s")

  @functools.partial(
      pl.kernel,
      out_type=jax.ShapeDtypeStruct((_N, _D), jnp.float32),
      mesh=mesh,
      compiler_params=pltpu.CompilerParams(use_tc_tiling_on_sc=False),
      scratch_types=[
          pltpu.VMEM((_NCH, _CHUNK), jnp.int32),
          pltpu.VMEM((2 * _L * _D,), jnp.float32),
      ] + [pltpu.VMEM((_CHUNK, _D), jnp.float32) for _ in range(_NSLOT)]
        + [pltpu.SemaphoreType.DMA for _ in range(2 * _NSLOT)],
  )
  def sc_embed(x_hbm, tok_hbm, pos_hbm, out_hbm, idx_v, pos_v, *bufs_sems):
    rows = bufs_sems[:_NSLOT]
    gsem = bufs_sems[_NSLOT:2 * _NSLOT]
    wsem = bufs_sems[2 * _NSLOT:]
    nc = 2
    wid = lax.axis_index("s") * nc + lax.axis_index("c")
    row_base = wid * _PER_W

    pltpu.sync_copy(x_hbm.at[pl.ds(wid * _NCH, _NCH)], idx_v)
    pltpu.sync_copy(pos_hbm, pos_v)

    def add_pos(rows_s, po):
      base = po * _D

      def row_body(i, _):
        off = base + i * _D
        for j in range(_D // 16):
          pv = pos_v[pl.ds(off + j * 16, 16)]
          plsc.addupdate(rows_s.at[i, pl.ds(j * 16, 16)], pv)
        return 0

      lax.fori_loop(0, _CHUNK, row_body, 0, unroll=8)

    for s in range(_LEAD):
      pltpu.async_copy(tok_hbm.at[idx_v.at[s]], rows[s], gsem[s])

    def group_body(g, po):
      for s0 in range(_NSLOT):
        c = g * _NSLOT + s0
        sl = (s0 + _LEAD) % _NSLOT
        cl = c + _LEAD

        @pl.when(cl < _NCH)
        def _():
          @pl.when(cl >= _NSLOT)
          def _():
            pltpu.make_async_copy(
                rows[sl],
                out_hbm.at[pl.ds(row_base + (cl - _NSLOT) * _CHUNK, _CHUNK)],
                wsem[sl]).wait()
          pltpu.async_copy(tok_hbm.at[idx_v.at[cl]], rows[sl], gsem[sl])

        pltpu.make_async_copy(tok_hbm.at[idx_v.at[c]], rows[s0],
                              gsem[s0]).wait()
        add_pos(rows[s0], po)
        pltpu.async_copy(rows[s0],
                         out_hbm.at[pl.ds(row_base + c * _CHUNK, _CHUNK)],
                         wsem[s0])
        po = po + _CHUNK
        po = lax.select(po >= _L, po - _L, po)
      return po

    lax.fori_loop(0, _NCH // _NSLOT, group_body, 0)

    for s in range(_NSLOT):
      c = _NCH - _NSLOT + s
      pltpu.make_async_copy(
          rows[s], out_hbm.at[pl.ds(row_base + c * _CHUNK, _CHUNK)],
          wsem[s]).wait()

  return sc_embed


_sc_embed = _make_sc_call()


@jax.jit
def kernel(x, tok_emb, pos_emb):
  x2d = x.astype(jnp.int32).reshape(_N // _CHUNK, _CHUNK)
  pos1 = pos_emb[:_L].reshape(-1)
  pos2 = jnp.concatenate([pos1, pos1])
  out = _sc_embed(x2d, tok_emb, pos2)
  return out.reshape(_B, _L, _D)


# speedup vs baseline: 1.2341x; 1.2212x over previous
import functools

import jax
import jax.numpy as jnp
from jax import lax
from jax.experimental import pallas as pl
from jax.experimental.pallas import tpu as pltpu
from jax.experimental.pallas import tpu_sc as plsc

_B = 4096
_L = 200
_D = 64
_NW = 32
_BPW = _B // _NW
_NSLOT = 4
_LEAD = 2


def _make_sc_call():
  mesh = plsc.VectorSubcoreMesh(core_axis_name="c", subcore_axis_name="s")

  @functools.partial(
      pl.kernel,
      out_type=jax.ShapeDtypeStruct((_B, _L, _D), jnp.float32),
      mesh=mesh,
      compiler_params=pltpu.CompilerParams(use_tc_tiling_on_sc=False),
      scratch_types=[
          pltpu.VMEM((_BPW, _L), jnp.int32),
          pltpu.VMEM((_L, _D), jnp.float32),
      ] + [pltpu.VMEM((_L, _D), jnp.float32) for _ in range(_NSLOT)]
        + [pltpu.SemaphoreType.DMA for _ in range(2 * _NSLOT)],
  )
  def sc_embed(x_hbm, tok_hbm, pos_hbm, out_hbm, idx_v, pos_v, *bufs_sems):
    rows = bufs_sems[:_NSLOT]
    gsem = bufs_sems[_NSLOT:2 * _NSLOT]
    wsem = bufs_sems[2 * _NSLOT:]
    nc = 2
    wid = lax.axis_index("s") * nc + lax.axis_index("c")
    b_base = wid * _BPW

    pltpu.sync_copy(x_hbm.at[pl.ds(b_base, _BPW)], idx_v)
    pltpu.sync_copy(pos_hbm.at[pl.ds(0, _L)], pos_v)

    def add_pos(rows_s):
      def row_body(i, _):
        for j in range(_D // 16):
          plsc.addupdate(rows_s.at[i, pl.ds(j * 16, 16)],
                         pos_v[i, pl.ds(j * 16, 16)])
        return 0

      lax.fori_loop(0, _L, row_body, 0, unroll=8)

    for s in range(_LEAD):
      pltpu.async_copy(tok_hbm.at[idx_v.at[s]], rows[s], gsem[s])

    def group_body(g, _):
      for s0 in range(_NSLOT):
        c = g * _NSLOT + s0
        sl = (s0 + _LEAD) % _NSLOT
        cl = c + _LEAD

        @pl.when(cl < _BPW)
        def _():
          @pl.when(cl >= _NSLOT)
          def _():
            pltpu.make_async_copy(
                rows[sl], out_hbm.at[b_base + cl - _NSLOT], wsem[sl]).wait()
          pltpu.async_copy(tok_hbm.at[idx_v.at[cl]], rows[sl], gsem[sl])

        pltpu.make_async_copy(tok_hbm.at[idx_v.at[c]], rows[s0],
                              gsem[s0]).wait()
        add_pos(rows[s0])
        pltpu.async_copy(rows[s0], out_hbm.at[b_base + c], wsem[s0])
      return 0

    lax.fori_loop(0, _BPW // _NSLOT, group_body, 0)

    for s in range(_NSLOT):
      c = _BPW - _NSLOT + s
      pltpu.make_async_copy(rows[s], out_hbm.at[b_base + c], wsem[s]).wait()

  return sc_embed


_sc_embed = _make_sc_call()


@jax.jit
def kernel(x, tok_emb, pos_emb):
  return _sc_embed(x.astype(jnp.int32), tok_emb, pos_emb)
